# baseline (speedup 1.0000x reference)
import functools

import jax
import jax.numpy as jnp
from jax import lax
from jax.experimental import pallas as pl
from jax.experimental.pallas import tpu as pltpu
from jax.experimental.pallas import tpu_sc as plsc

N = 10000
E = 320000
D = 128
EPS = 1e-5

NC = 2
NS = 16
CK = 125
EPT = E // (NC * NS)
NCHUNK = EPT // CK
CKA = 125
NCHA = EPT // CKA
NODE_CH = 640
NODE_LAST = N - NODE_CH * (NS - 1)
NP = NODE_CH * NS

_mesh = plsc.VectorSubcoreMesh(
    core_axis_name="c", subcore_axis_name="s", num_cores=NC, num_subcores=NS
)


@functools.partial(
    pl.kernel,
    out_type=jax.ShapeDtypeStruct((NC, 1, NP), jnp.float32),
    mesh=_mesh,
    scratch_types=[
        pltpu.VMEM((NCHUNK, CK), jnp.int32),
        pltpu.VMEM((CK,), jnp.float32),
        pltpu.VMEM((NODE_CH,), jnp.float32),
        pltpu.VMEM_SHARED((NP,), jnp.float32),
    ],
)
def _deg_sc(dst_hbm, degp_hbm, dstbuf, ones, zb, hist):
    c = lax.axis_index("c")
    s = lax.axis_index("s")
    for i in range(NODE_CH // 16):
        zb[pl.ds(16 * i, 16)] = jnp.zeros((16,), jnp.float32)
    for i in range(CK // 16):
        ones[pl.ds(16 * i, 16)] = jnp.ones((16,), jnp.float32)
    if CK % 16:
        ones[pl.ds(CK - 16, 16)] = jnp.ones((16,), jnp.float32)

    pltpu.sync_copy(zb, hist.at[pl.ds(s * NODE_CH, NODE_CH)])
    pltpu.sync_copy(dst_hbm.at[c * NS + s], dstbuf)
    plsc.subcore_barrier()

    def body(j, carry):
        pltpu.sync_copy(ones, hist.at[dstbuf.at[j]], add=True)
        return carry

    lax.fori_loop(0, NCHUNK, body, 0)
    plsc.subcore_barrier()

    pltpu.sync_copy(
        hist.at[pl.ds(s * NODE_CH, NODE_CH)],
        degp_hbm.at[c, 0, pl.ds(s * NODE_CH, NODE_CH)],
    )


@functools.partial(
    pl.kernel,
    out_type=jax.ShapeDtypeStruct((NC, N, D), jnp.float32),
    mesh=_mesh,
    scratch_types=[
        pltpu.VMEM((2, NCHA // 2, CKA), jnp.int32),
        pltpu.VMEM((2, CKA, D), jnp.float32),
        pltpu.VMEM_SHARED((N, D), jnp.float32),
        pltpu.SemaphoreType.DMA,
        pltpu.SemaphoreType.DMA,
    ],
)
def _accum_sc(g_hbm, edge_hbm, acc_hbm, ebuf, rows, accum, sem0, sem1):
    c = lax.axis_index("c")
    s = lax.axis_index("s")

    def zbody(i, carry):
        r = i // 8
        l = i % 8
        rows[0, r, pl.ds(16 * l, 16)] = jnp.zeros((16,), jnp.float32)
        return carry

    lax.fori_loop(0, CKA * 8, zbody, 0)

    base = s * NODE_CH
    for k in range(NODE_CH // CKA):
        pltpu.sync_copy(rows.at[0], accum.at[pl.ds(base + CKA * k, CKA)])
    pltpu.sync_copy(
        rows.at[0, pl.ds(0, 16)],
        accum.at[pl.ds(base + NODE_CH - 16, 16)],
    )

    wid = c * NS + s
    plsc.subcore_barrier()

    HC = NCHA // 2

    for half in range(2):
        pltpu.sync_copy(edge_hbm.at[wid, half], ebuf)
        pltpu.async_copy(g_hbm.at[ebuf.at[0, 0]], rows.at[0], sem0)

        def body(jj, carry):
            j = 2 * jj
            pltpu.async_copy(g_hbm.at[ebuf.at[0, j + 1]], rows.at[1], sem1)
            pltpu.make_async_copy(g_hbm.at[ebuf.at[0, j]], rows.at[0], sem0).wait()
            pltpu.sync_copy(rows.at[0], accum.at[ebuf.at[1, j]], add=True)

            @pl.when(jj < HC // 2 - 1)
            def _():
                pltpu.async_copy(g_hbm.at[ebuf.at[0, j + 2]], rows.at[0], sem0)

            pltpu.make_async_copy(g_hbm.at[ebuf.at[0, j + 1]], rows.at[1], sem1).wait()
            pltpu.sync_copy(rows.at[1], accum.at[ebuf.at[1, j + 1]], add=True)
            return carry

        lax.fori_loop(0, HC // 2, body, 0)

    plsc.subcore_barrier()

    @pl.when(s < NS - 1)
    def _():
        pltpu.sync_copy(
            accum.at[pl.ds(s * NODE_CH, NODE_CH)],
            acc_hbm.at[c, pl.ds(s * NODE_CH, NODE_CH)],
        )

    @pl.when(s == NS - 1)
    def _():
        pltpu.sync_copy(
            accum.at[pl.ds((NS - 1) * NODE_CH, NODE_LAST)],
            acc_hbm.at[c, pl.ds((NS - 1) * NODE_CH, NODE_LAST)],
        )


BR = 2000


def _dense1_body(x_ref, w_ref, degp_ref, g_ref, dinv_ref):
    deg = degp_ref[0] + degp_ref[1] + 1.0
    dinv = lax.rsqrt(deg)
    h = jnp.dot(
        x_ref[...], w_ref[...],
        preferred_element_type=jnp.float32, precision=lax.Precision.HIGHEST,
    )
    g_ref[...] = h * dinv
    dinv_ref[...] = dinv


_dense1 = pl.pallas_call(
    _dense1_body,
    grid=(N // BR,),
    in_specs=[
        pl.BlockSpec((BR, D), lambda i: (i, 0)),
        pl.BlockSpec((D, D), lambda i: (0, 0)),
        pl.BlockSpec((NC, BR, 1), lambda i: (0, i, 0)),
    ],
    out_specs=[
        pl.BlockSpec((BR, D), lambda i: (i, 0)),
        pl.BlockSpec((BR, 1), lambda i: (i, 0)),
    ],
    out_shape=[
        jax.ShapeDtypeStruct((N, D), jnp.float32),
        jax.ShapeDtypeStruct((N, 1), jnp.float32),
    ],
)


def _dense2_body(acc_ref, g_ref, dinv_ref, b_ref, gam_ref, bet_ref, x_ref, o_ref):
    pre = (acc_ref[0] + acc_ref[1] + g_ref[...]) * dinv_ref[...] + b_ref[...]
    mu = jnp.mean(pre, axis=-1, keepdims=True)
    ctr = pre - mu
    var = jnp.mean(ctr * ctr, axis=-1, keepdims=True)
    xh = ctr * lax.rsqrt(var + EPS)
    y = xh * gam_ref[...] + bet_ref[...] + x_ref[...]
    o_ref[...] = jnp.maximum(y, 0.0)


_dense2 = pl.pallas_call(
    _dense2_body,
    grid=(N // BR,),
    in_specs=[
        pl.BlockSpec((NC, BR, D), lambda i: (0, i, 0)),
        pl.BlockSpec((BR, D), lambda i: (i, 0)),
        pl.BlockSpec((BR, 1), lambda i: (i, 0)),
        pl.BlockSpec((1, D), lambda i: (0, 0)),
        pl.BlockSpec((1, D), lambda i: (0, 0)),
        pl.BlockSpec((1, D), lambda i: (0, 0)),
        pl.BlockSpec((BR, D), lambda i: (i, 0)),
    ],
    out_specs=pl.BlockSpec((BR, D), lambda i: (i, 0)),
    out_shape=jax.ShapeDtypeStruct((N, D), jnp.float32),
)


def kernel(x, edge_index, W, b, gamma, beta):
    dst_rs = edge_index[1].reshape(NC * NS, NCHUNK, CK)
    edge_rs = jnp.transpose(
        edge_index.reshape(2, NC * NS, 2, NCHA // 2, CKA), (1, 2, 0, 3, 4)
    )
    degp = _deg_sc(dst_rs)
    g, dinv = _dense1(x, W, degp[:, 0, :N].reshape(NC, N, 1))
    acc = _accum_sc(g, edge_rs)
    out = _dense2(
        acc, g, dinv,
        b.reshape(1, D), gamma.reshape(1, D), beta.reshape(1, D), x,
    )
    return out


# speedup vs baseline: 1.0038x; 1.0038x over previous
import functools

import jax
import jax.numpy as jnp
from jax import lax
from jax.experimental import pallas as pl
from jax.experimental.pallas import tpu as pltpu
from jax.experimental.pallas import tpu_sc as plsc

N = 10000
E = 320000
D = 128
EPS = 1e-5

NC = 2
NS = 16
CK = 100
EPT = E // (NC * NS)
NCHUNK = EPT // CK
CKA = 125
NCHA = EPT // CKA
NODE_CH = 640
NODE_LAST = N - NODE_CH * (NS - 1)
NP = NODE_CH * NS

_mesh = plsc.VectorSubcoreMesh(
    core_axis_name="c", subcore_axis_name="s", num_cores=NC, num_subcores=NS
)


@functools.partial(
    pl.kernel,
    out_type=jax.ShapeDtypeStruct((NC, 1, NP), jnp.float32),
    mesh=_mesh,
    scratch_types=[
        pltpu.VMEM((NCHUNK, CK), jnp.int32),
        pltpu.VMEM((CK,), jnp.float32),
        pltpu.VMEM((NODE_CH,), jnp.float32),
        pltpu.VMEM_SHARED((NP,), jnp.float32),
    ],
)
def _deg_sc(dst_hbm, degp_hbm, dstbuf, ones, zb, hist):
    c = lax.axis_index("c")
    s = lax.axis_index("s")
    for i in range(NODE_CH // 16):
        zb[pl.ds(16 * i, 16)] = jnp.zeros((16,), jnp.float32)
    for i in range(CK // 16):
        ones[pl.ds(16 * i, 16)] = jnp.ones((16,), jnp.float32)
    if CK % 16:
        ones[pl.ds(CK - 16, 16)] = jnp.ones((16,), jnp.float32)

    pltpu.sync_copy(zb, hist.at[pl.ds(s * NODE_CH, NODE_CH)])
    pltpu.sync_copy(dst_hbm.at[c * NS + s], dstbuf)
    plsc.subcore_barrier()

    def body(j, carry):
        pltpu.sync_copy(ones, hist.at[dstbuf.at[j]], add=True)
        return carry

    lax.fori_loop(0, NCHUNK, body, 0)
    plsc.subcore_barrier()

    pltpu.sync_copy(
        hist.at[pl.ds(s * NODE_CH, NODE_CH)],
        degp_hbm.at[c, 0, pl.ds(s * NODE_CH, NODE_CH)],
    )


@functools.partial(
    pl.kernel,
    out_type=jax.ShapeDtypeStruct((NC, N, D), jnp.float32),
    mesh=_mesh,
    scratch_types=[
        pltpu.VMEM((2, NCHA // 2, CKA), jnp.int32),
        pltpu.VMEM((2, CKA, D), jnp.float32),
        pltpu.VMEM_SHARED((N, D), jnp.float32),
        pltpu.SemaphoreType.DMA,
        pltpu.SemaphoreType.DMA,
    ],
)
def _accum_sc(g_hbm, edge_hbm, acc_hbm, ebuf, rows, accum, sem0, sem1):
    c = lax.axis_index("c")
    s = lax.axis_index("s")

    def zbody(i, carry):
        r = i // 8
        l = i % 8
        rows[0, r, pl.ds(16 * l, 16)] = jnp.zeros((16,), jnp.float32)
        return carry

    lax.fori_loop(0, CKA * 8, zbody, 0)

    base = s * NODE_CH
    for k in range(NODE_CH // CKA):
        pltpu.sync_copy(rows.at[0], accum.at[pl.ds(base + CKA * k, CKA)])
    pltpu.sync_copy(
        rows.at[0, pl.ds(0, 16)],
        accum.at[pl.ds(base + NODE_CH - 16, 16)],
    )

    wid = c * NS + s
    plsc.subcore_barrier()

    HC = NCHA // 2

    for half in range(2):
        pltpu.sync_copy(edge_hbm.at[wid, half], ebuf)
        pltpu.async_copy(g_hbm.at[ebuf.at[0, 0]], rows.at[0], sem0)

        def body(jj, carry):
            j = 2 * jj
            pltpu.async_copy(g_hbm.at[ebuf.at[0, j + 1]], rows.at[1], sem1)
            pltpu.make_async_copy(g_hbm.at[ebuf.at[0, j]], rows.at[0], sem0).wait()
            pltpu.sync_copy(rows.at[0], accum.at[ebuf.at[1, j]], add=True)

            @pl.when(jj < HC // 2 - 1)
            def _():
                pltpu.async_copy(g_hbm.at[ebuf.at[0, j + 2]], rows.at[0], sem0)

            pltpu.make_async_copy(g_hbm.at[ebuf.at[0, j + 1]], rows.at[1], sem1).wait()
            pltpu.sync_copy(rows.at[1], accum.at[ebuf.at[1, j + 1]], add=True)
            return carry

        lax.fori_loop(0, HC // 2, body, 0)

    plsc.subcore_barrier()

    @pl.when(s < NS - 1)
    def _():
        pltpu.sync_copy(
            accum.at[pl.ds(s * NODE_CH, NODE_CH)],
            acc_hbm.at[c, pl.ds(s * NODE_CH, NODE_CH)],
        )

    @pl.when(s == NS - 1)
    def _():
        pltpu.sync_copy(
            accum.at[pl.ds((NS - 1) * NODE_CH, NODE_LAST)],
            acc_hbm.at[c, pl.ds((NS - 1) * NODE_CH, NODE_LAST)],
        )


BR = 2000


def _dense1_body(x_ref, w_ref, degp_ref, g_ref, dinv_ref):
    deg = degp_ref[0] + degp_ref[1] + 1.0
    dinv = lax.rsqrt(deg)
    h = jnp.dot(
        x_ref[...], w_ref[...],
        preferred_element_type=jnp.float32, precision=lax.Precision.HIGHEST,
    )
    g_ref[...] = h * dinv
    dinv_ref[...] = dinv


_dense1 = pl.pallas_call(
    _dense1_body,
    grid=(N // BR,),
    in_specs=[
        pl.BlockSpec((BR, D), lambda i: (i, 0)),
        pl.BlockSpec((D, D), lambda i: (0, 0)),
        pl.BlockSpec((NC, BR, 1), lambda i: (0, i, 0)),
    ],
    out_specs=[
        pl.BlockSpec((BR, D), lambda i: (i, 0)),
        pl.BlockSpec((BR, 1), lambda i: (i, 0)),
    ],
    out_shape=[
        jax.ShapeDtypeStruct((N, D), jnp.float32),
        jax.ShapeDtypeStruct((N, 1), jnp.float32),
    ],
)


def _dense2_body(acc_ref, g_ref, dinv_ref, b_ref, gam_ref, bet_ref, x_ref, o_ref):
    pre = (acc_ref[0] + acc_ref[1] + g_ref[...]) * dinv_ref[...] + b_ref[...]
    mu = jnp.mean(pre, axis=-1, keepdims=True)
    ctr = pre - mu
    var = jnp.mean(ctr * ctr, axis=-1, keepdims=True)
    xh = ctr * lax.rsqrt(var + EPS)
    y = xh * gam_ref[...] + bet_ref[...] + x_ref[...]
    o_ref[...] = jnp.maximum(y, 0.0)


_dense2 = pl.pallas_call(
    _dense2_body,
    grid=(N // BR,),
    in_specs=[
        pl.BlockSpec((NC, BR, D), lambda i: (0, i, 0)),
        pl.BlockSpec((BR, D), lambda i: (i, 0)),
        pl.BlockSpec((BR, 1), lambda i: (i, 0)),
        pl.BlockSpec((1, D), lambda i: (0, 0)),
        pl.BlockSpec((1, D), lambda i: (0, 0)),
        pl.BlockSpec((1, D), lambda i: (0, 0)),
        pl.BlockSpec((BR, D), lambda i: (i, 0)),
    ],
    out_specs=pl.BlockSpec((BR, D), lambda i: (i, 0)),
    out_shape=jax.ShapeDtypeStruct((N, D), jnp.float32),
)


def kernel(x, edge_index, W, b, gamma, beta):
    dst_rs = edge_index[1].reshape(NC * NS, NCHUNK, CK)
    edge_rs = jnp.transpose(
        edge_index.reshape(2, NC * NS, 2, NCHA // 2, CKA), (1, 2, 0, 3, 4)
    )
    degp = _deg_sc(dst_rs)
    g, dinv = _dense1(x, W, degp[:, 0, :N].reshape(NC, N, 1))
    acc = _accum_sc(g, edge_rs)
    out = _dense2(
        acc, g, dinv,
        b.reshape(1, D), gamma.reshape(1, D), beta.reshape(1, D), x,
    )
    return out
